# baseline (speedup 1.0000x reference)
import functools

import jax
import jax.numpy as jnp
from jax import lax
from jax.experimental import pallas as pl
from jax.experimental.pallas import tpu as pltpu
from jax.experimental.pallas import tpu_sc as plsc

B, S = 4, 8192
D = 1024
NT = 8
N = B * S
NC, NS = 2, 16
NW = NC * NS
TOK_PER_W = N // NW
C = 16
NCHUNK = TOK_PER_W // C

_mesh = plsc.VectorSubcoreMesh(core_axis_name="c", subcore_axis_name="s")


@functools.partial(
    pl.kernel,
    mesh=_mesh,
    compiler_params=pltpu.CompilerParams(needs_layout_passes=False),
    out_type=jax.ShapeDtypeStruct((N, D), jnp.float32),
    scratch_types=[
        pltpu.VMEM((C,), jnp.int32),
        pltpu.VMEM((C,), jnp.int32),
        pltpu.VMEM((NT * D,), jnp.float32),
        pltpu.VMEM((C, D), jnp.float32),
        pltpu.SemaphoreType.DMA,
    ],
)
def _emb(ids_hbm, cids_hbm, tok_tbl_hbm, code_tbl_hbm, out_hbm,
         idx_v, cidx_v, ctbl_v, tok_v, sem):
    wid = lax.axis_index("s") * NC + lax.axis_index("c")
    base = wid * TOK_PER_W

    pltpu.sync_copy(code_tbl_hbm, ctbl_v)
    iota16 = lax.iota(jnp.int32, 16)

    def _chunk(c, carry):
        cb = base + c * C
        pltpu.sync_copy(ids_hbm.at[pl.ds(cb, C)], idx_v)
        pltpu.sync_copy(cids_hbm.at[pl.ds(cb, C)], cidx_v)
        pltpu.async_copy(tok_tbl_hbm.at[idx_v], tok_v, sem).wait()

        NSEG = D // 16
        DEPTH = 8
        cvec = cidx_v[pl.ds(0, 16)]
        for r in range(C):
            spl = jnp.full((16,), r, jnp.int32)
            cid_s = lax.reduce_max(jnp.where(iota16 == r, cvec, 0), (0,))
            cbase = lax.shift_left(jnp.zeros((16,), jnp.int32) + cid_s, 10)
            cols = [iota16 + j * 16 for j in range(NSEG)]
            ring = [plsc.load_gather(ctbl_v, [cbase + cols[j]])
                    for j in range(DEPTH)]
            for j in range(DEPTH, NSEG):
                plsc.addupdate_scatter(tok_v, [spl, cols[j - DEPTH]],
                                       ring[j % DEPTH])
                ring[j % DEPTH] = plsc.load_gather(ctbl_v, [cbase + cols[j]])
            for j in range(NSEG - DEPTH, NSEG):
                plsc.addupdate_scatter(tok_v, [spl, cols[j]], ring[j % DEPTH])
        pltpu.sync_copy(tok_v, out_hbm.at[pl.ds(cb, C)])
        return carry

    lax.fori_loop(0, NCHUNK, _chunk, 0)


def kernel(input_ids, code_type_ids, token_table, code_table):
    ids = input_ids.reshape(N).astype(jnp.int32)
    cids = code_type_ids.reshape(N).astype(jnp.int32)
    out = _emb(ids, cids, token_table, code_table.reshape(NT * D))
    return out.reshape(B, S, D)
